# Initial kernel scaffold; baseline (speedup 1.0000x reference)
import jax
import jax.numpy as jnp
from jax.experimental import pallas as pl


def kernel(x, edge_index, W1, b1, W2, b2, W3, b3, W4, b4, W5, b5, W6, b6, g1, be1, g2, be2, g3, be3, g4, be4, g5, be5):
    raise NotImplementedError("write your pallas kernel here")


# speedup vs baseline: 5.0474x; 5.0474x over previous
import functools

import jax
import jax.numpy as jnp
from jax import lax
from jax.experimental import pallas as pl
from jax.experimental.pallas import tpu as pltpu
from jax.experimental.pallas import tpu_sc as plsc

N = 10000
F32 = jnp.float32
EPS = 1e-5
NC, NS = 2, 16
RPT = 632
NACC = NS * RPT
RPT_LAST = N - (NS - 1) * RPT
DUMP_ROW = 10008
E_RAW = 320000
K = 128
EC = 2560
E_PAD = EC * K
RB = 2000
NRB = N // RB

_MESH = plsc.VectorSubcoreMesh(
    core_axis_name="c", subcore_axis_name="s", num_cores=NC, num_subcores=NS)


def _make_agg(split_edges):
  nch = EC // (NC * NS) if split_edges else EC // NS
  IG = 8

  @functools.partial(
      pl.kernel,
      out_type=jax.ShapeDtypeStruct((NC, N, 128), F32),
      mesh=_MESH,
      scratch_types=[
          pltpu.VMEM((IG, 2, K), jnp.int32),
          pltpu.VMEM((K, 128), F32),
          pltpu.VMEM_SHARED((NACC, 128), F32),
          pltpu.SemaphoreType.DMA,
      ],
  )
  def agg(m, ed3, zeros, out, ed_v, stage, acc, sem):
    c = lax.axis_index("c")
    s = lax.axis_index("s")
    row0 = s * RPT
    pltpu.sync_copy(zeros, acc.at[pl.ds(row0, RPT)])
    if split_edges:
      cb = (c * NS + s) * nch
      g = c * 0
    else:
      cb = s * nch
      g = c
    plsc.subcore_barrier()

    @pl.loop(0, nch // IG)
    def _(jj):
      pltpu.sync_copy(ed3.at[pl.ds(cb + jj * IG, IG)], ed_v)
      for i in range(IG):
        pltpu.async_copy(m.at[g].at[ed_v.at[i, 0]], stage, sem).wait()
        pltpu.sync_copy(stage, acc.at[ed_v.at[i, 1]], add=True)

    plsc.subcore_barrier()

    @pl.when(s < NS - 1)
    def _():
      pltpu.sync_copy(acc.at[pl.ds(row0, RPT)], out.at[c, pl.ds(row0, RPT)])

    @pl.when(s == NS - 1)
    def _():
      pltpu.sync_copy(acc.at[pl.ds(row0, RPT_LAST)],
                      out.at[c, pl.ds(row0, RPT_LAST)])

  return agg


_agg_half = _make_agg(False)
_agg_full = _make_agg(True)

def _dinv_of(d_ref):
  deg = d_ref[0][:, :1] + d_ref[1][:, :1]
  return lax.rsqrt(deg + 1.0)


def _dot(a, b):
  return jnp.dot(a, b, preferred_element_type=F32,
                 precision=lax.Precision.HIGHEST)


def _m1_body(x_ref, w_ref, d_ref, o_ref):
  dinv = _dinv_of(d_ref)
  o_ref[...] = (_dot(x_ref[...], w_ref[...]) * dinv)[None]


_m1 = pl.pallas_call(
    _m1_body,
    grid=(2, NRB),
    in_specs=[
        pl.BlockSpec((RB, 128), lambda c, r: (r, 0)),
        pl.BlockSpec((128, 128), lambda c, r: (0, c)),
        pl.BlockSpec((2, RB, 16), lambda c, r: (0, r, 0)),
    ],
    out_specs=pl.BlockSpec((1, RB, 128), lambda c, r: (c, r, 0)),
    out_shape=jax.ShapeDtypeStruct((2, N, 128), F32),
)


def _stats_body(s_ref, mp_ref, d_ref, b_ref, o_ref):
  r = pl.program_id(0)
  dinv = _dinv_of(d_ref)
  sums, sqs = [], []
  for ch in range(2):
    z = dinv * (s_ref[ch] + mp_ref[ch]) + b_ref[ch]
    sums.append(jnp.sum(z, axis=0))
    sqs.append(jnp.sum(z * z, axis=0))
  sm = jnp.stack(sums)
  sq = jnp.stack(sqs)

  @pl.when(r == 0)
  def _():
    o_ref[0] = sm
    o_ref[1] = sq

  @pl.when(r > 0)
  def _():
    o_ref[0] += sm
    o_ref[1] += sq

  @pl.when(r == NRB - 1)
  def _():
    mean = o_ref[0] / N
    o_ref[0] = mean
    o_ref[1] = o_ref[1] / N - mean * mean


_stats = pl.pallas_call(
    _stats_body,
    grid=(NRB,),
    in_specs=[
        pl.BlockSpec((2, RB, 128), lambda r: (0, r, 0)),
        pl.BlockSpec((2, RB, 128), lambda r: (0, r, 0)),
        pl.BlockSpec((2, RB, 16), lambda r: (0, r, 0)),
        pl.BlockSpec((2, 128), lambda r: (0, 0)),
    ],
    out_specs=pl.BlockSpec((2, 2, 128), lambda r: (0, 0, 0)),
    out_shape=jax.ShapeDtypeStruct((2, 2, 128), F32),
)


def _fused_body(s_ref, mp_ref, d_ref, b_ref, st_ref, g_ref, be_ref, w_ref,
                o_ref):
  dinv = _dinv_of(d_ref)
  w = w_ref[0]
  acc = None
  for ch in range(2):
    z = dinv * (s_ref[ch] + mp_ref[ch]) + b_ref[ch]
    alpha = lax.rsqrt(st_ref[1, ch] + EPS) * g_ref[ch]
    h = jnp.maximum((z - st_ref[0, ch]) * alpha + be_ref[ch], 0.0) * dinv
    p = _dot(h, w[ch * 128:(ch + 1) * 128])
    acc = p if acc is None else acc + p
  o_ref[...] = acc[None]


def _make_fused(h_out):
  return pl.pallas_call(
      _fused_body,
      grid=(h_out, NRB),
      in_specs=[
          pl.BlockSpec((2, RB, 128), lambda c, r: (0, r, 0)),
          pl.BlockSpec((2, RB, 128), lambda c, r: (0, r, 0)),
          pl.BlockSpec((2, RB, 16), lambda c, r: (0, r, 0)),
          pl.BlockSpec((2, 128), lambda c, r: (0, 0)),
          pl.BlockSpec((2, 2, 128), lambda c, r: (0, 0, 0)),
          pl.BlockSpec((2, 128), lambda c, r: (0, 0)),
          pl.BlockSpec((2, 128), lambda c, r: (0, 0)),
          pl.BlockSpec((1, 256, 128), lambda c, r: (c, 0, 0)),
      ],
      out_specs=pl.BlockSpec((1, RB, 128), lambda c, r: (c, r, 0)),
      out_shape=jax.ShapeDtypeStruct((h_out, N, 128), F32),
  )


_fused2 = _make_fused(2)
_fused1 = _make_fused(1)


def _final_body(s_ref, m6_ref, d_ref, b_ref, o_ref):
  dinv = _dinv_of(d_ref)
  o_ref[...] = dinv * (s_ref[0] + s_ref[1] + m6_ref[0]) + b_ref[...]


_final = pl.pallas_call(
    _final_body,
    grid=(NRB,),
    in_specs=[
        pl.BlockSpec((2, RB, 128), lambda r: (0, r, 0)),
        pl.BlockSpec((1, RB, 128), lambda r: (0, r, 0)),
        pl.BlockSpec((2, RB, 16), lambda r: (0, r, 0)),
        pl.BlockSpec((1, 128), lambda r: (0, 0)),
    ],
    out_specs=pl.BlockSpec((RB, 128), lambda r: (r, 0)),
    out_shape=jax.ShapeDtypeStruct((N, 128), F32),
)


def kernel(x, edge_index, W1, b1, W2, b2, W3, b3, W4, b4, W5, b5, W6, b6,
           g1, be1, g2, be2, g3, be3, g4, be4, g5, be5):
  ei = edge_index.astype(jnp.int32)
  npad = E_PAD - E_RAW
  src2 = jnp.concatenate(
      [ei[0], jnp.zeros((npad,), jnp.int32)]).reshape(EC, 1, K)
  dst2 = jnp.concatenate(
      [ei[1], jnp.full((npad,), DUMP_ROW, jnp.int32)]).reshape(EC, 1, K)
  ed3 = jnp.concatenate([src2, dst2], axis=1)
  zeros128 = jnp.zeros((RPT, 128), F32)
  deg16 = _agg_full(jnp.ones((1, N, 128), F32), ed3, zeros128)[:, :, :16]

  two = lambda v: v.reshape(2, 128)
  w_halves = lambda W: W.reshape(256, -1, 128).transpose(1, 0, 2)

  bs = [two(b1), two(b2), two(b3), two(b4), two(b5)]
  gs = [two(g1), two(g2), two(g3), two(g4), two(g5)]
  bes = [two(be1), two(be2), two(be3), two(be4), two(be5)]
  Ws = [w_halves(W2), w_halves(W3), w_halves(W4), w_halves(W5), w_halves(W6)]

  M = _m1(x, W1, deg16)
  for l in range(5):
    S = _agg_half(M, ed3, zeros128)
    st = _stats(S, M, deg16, bs[l])
    fused = _fused2 if l < 4 else _fused1
    M = fused(S, M, deg16, bs[l], st, gs[l], bes[l], Ws[l])
  S6 = _agg_full(M, ed3, zeros128)
  return _final(S6, M, deg16, b6.reshape(1, 128))


# speedup vs baseline: 5.9524x; 1.1793x over previous
import functools

import jax
import jax.numpy as jnp
from jax import lax
from jax.experimental import pallas as pl
from jax.experimental.pallas import tpu as pltpu
from jax.experimental.pallas import tpu_sc as plsc

N = 10000
F32 = jnp.float32
EPS = 1e-5
NC, NS = 2, 16
RPT = 632
NACC = NS * RPT
RPT_LAST = N - (NS - 1) * RPT
DUMP_ROW = 10008
E_RAW = 320000
K = 128
EC = 2560
E_PAD = EC * K
RB = 2000
NRB = N // RB

_MESH = plsc.VectorSubcoreMesh(
    core_axis_name="c", subcore_axis_name="s", num_cores=NC, num_subcores=NS)


def _make_agg(split_edges):
  nch = EC // (NC * NS) if split_edges else EC // NS
  IG = 16

  @functools.partial(
      pl.kernel,
      out_type=jax.ShapeDtypeStruct((NC, N, 128), F32),
      mesh=_MESH,
      scratch_types=[
          pltpu.VMEM((IG, 2, K), jnp.int32),
          pltpu.VMEM((K, 128), F32),
          pltpu.VMEM((K, 128), F32),
          pltpu.VMEM_SHARED((NACC, 128), F32),
          pltpu.SemaphoreType.DMA,
          pltpu.SemaphoreType.DMA,
          pltpu.SemaphoreType.DMA,
          pltpu.SemaphoreType.DMA,
      ],
  )
  def agg(m, ed3, zeros, out, ed_v, st0, st1, acc, gs0, gs1, ss0, ss1):
    c = lax.axis_index("c")
    s = lax.axis_index("s")
    row0 = s * RPT
    pltpu.sync_copy(zeros, acc.at[pl.ds(row0, RPT)])
    if split_edges:
      cb = (c * NS + s) * nch
      g = c * 0
    else:
      cb = s * nch
      g = c
    plsc.subcore_barrier()

    bufs = [(st0, gs0, ss0), (st1, gs1, ss1)]

    @pl.loop(0, nch // IG)
    def _(jj):
      pltpu.sync_copy(ed3.at[pl.ds(cb + jj * IG, IG)], ed_v)
      gd = [None] * IG
      for i in range(2):
        st, gs, _ = bufs[i]
        gd[i] = pltpu.async_copy(m.at[g].at[ed_v.at[i, 0]], st, gs)
      tail = []
      for i in range(IG):
        st, gs, ss = bufs[i % 2]
        gd[i].wait()
        sd = pltpu.async_copy(st, acc.at[ed_v.at[i, 1]], ss, add=True)
        if i + 2 < IG:
          sd.wait()
          gd[i + 2] = pltpu.async_copy(m.at[g].at[ed_v.at[i + 2, 0]], st, gs)
        else:
          tail.append(sd)
      for sd in tail:
        sd.wait()

    plsc.subcore_barrier()

    @pl.when(s < NS - 1)
    def _():
      pltpu.sync_copy(acc.at[pl.ds(row0, RPT)], out.at[c, pl.ds(row0, RPT)])

    @pl.when(s == NS - 1)
    def _():
      pltpu.sync_copy(acc.at[pl.ds(row0, RPT_LAST)],
                      out.at[c, pl.ds(row0, RPT_LAST)])

  return agg


_agg_half = _make_agg(False)
_agg_full = _make_agg(True)

def _dinv_of(d_ref):
  deg = d_ref[0][:, :1] + d_ref[1][:, :1]
  return lax.rsqrt(deg + 1.0)


def _dot(a, b):
  return jnp.dot(a, b, preferred_element_type=F32,
                 precision=lax.Precision.HIGHEST)


def _m1_body(x_ref, w_ref, d_ref, o_ref):
  dinv = _dinv_of(d_ref)
  o_ref[...] = (_dot(x_ref[...], w_ref[...]) * dinv)[None]


_m1 = pl.pallas_call(
    _m1_body,
    grid=(2, NRB),
    in_specs=[
        pl.BlockSpec((RB, 128), lambda c, r: (r, 0)),
        pl.BlockSpec((128, 128), lambda c, r: (0, c)),
        pl.BlockSpec((2, RB, 16), lambda c, r: (0, r, 0)),
    ],
    out_specs=pl.BlockSpec((1, RB, 128), lambda c, r: (c, r, 0)),
    out_shape=jax.ShapeDtypeStruct((2, N, 128), F32),
)


def _stats_body(s_ref, mp_ref, d_ref, b_ref, o_ref):
  r = pl.program_id(0)
  dinv = _dinv_of(d_ref)
  sums, sqs = [], []
  for ch in range(2):
    z = dinv * (s_ref[ch] + mp_ref[ch]) + b_ref[ch]
    sums.append(jnp.sum(z, axis=0))
    sqs.append(jnp.sum(z * z, axis=0))
  sm = jnp.stack(sums)
  sq = jnp.stack(sqs)

  @pl.when(r == 0)
  def _():
    o_ref[0] = sm
    o_ref[1] = sq

  @pl.when(r > 0)
  def _():
    o_ref[0] += sm
    o_ref[1] += sq

  @pl.when(r == NRB - 1)
  def _():
    mean = o_ref[0] / N
    o_ref[0] = mean
    o_ref[1] = o_ref[1] / N - mean * mean


_stats = pl.pallas_call(
    _stats_body,
    grid=(NRB,),
    in_specs=[
        pl.BlockSpec((2, RB, 128), lambda r: (0, r, 0)),
        pl.BlockSpec((2, RB, 128), lambda r: (0, r, 0)),
        pl.BlockSpec((2, RB, 16), lambda r: (0, r, 0)),
        pl.BlockSpec((2, 128), lambda r: (0, 0)),
    ],
    out_specs=pl.BlockSpec((2, 2, 128), lambda r: (0, 0, 0)),
    out_shape=jax.ShapeDtypeStruct((2, 2, 128), F32),
)


def _fused_body(s_ref, mp_ref, d_ref, b_ref, st_ref, g_ref, be_ref, w_ref,
                o_ref):
  dinv = _dinv_of(d_ref)
  w = w_ref[0]
  acc = None
  for ch in range(2):
    z = dinv * (s_ref[ch] + mp_ref[ch]) + b_ref[ch]
    alpha = lax.rsqrt(st_ref[1, ch] + EPS) * g_ref[ch]
    h = jnp.maximum((z - st_ref[0, ch]) * alpha + be_ref[ch], 0.0) * dinv
    p = _dot(h, w[ch * 128:(ch + 1) * 128])
    acc = p if acc is None else acc + p
  o_ref[...] = acc[None]


def _make_fused(h_out):
  return pl.pallas_call(
      _fused_body,
      grid=(h_out, NRB),
      in_specs=[
          pl.BlockSpec((2, RB, 128), lambda c, r: (0, r, 0)),
          pl.BlockSpec((2, RB, 128), lambda c, r: (0, r, 0)),
          pl.BlockSpec((2, RB, 16), lambda c, r: (0, r, 0)),
          pl.BlockSpec((2, 128), lambda c, r: (0, 0)),
          pl.BlockSpec((2, 2, 128), lambda c, r: (0, 0, 0)),
          pl.BlockSpec((2, 128), lambda c, r: (0, 0)),
          pl.BlockSpec((2, 128), lambda c, r: (0, 0)),
          pl.BlockSpec((1, 256, 128), lambda c, r: (c, 0, 0)),
      ],
      out_specs=pl.BlockSpec((1, RB, 128), lambda c, r: (c, r, 0)),
      out_shape=jax.ShapeDtypeStruct((h_out, N, 128), F32),
  )


_fused2 = _make_fused(2)
_fused1 = _make_fused(1)


def _final_body(s_ref, m6_ref, d_ref, b_ref, o_ref):
  dinv = _dinv_of(d_ref)
  o_ref[...] = dinv * (s_ref[0] + s_ref[1] + m6_ref[0]) + b_ref[...]


_final = pl.pallas_call(
    _final_body,
    grid=(NRB,),
    in_specs=[
        pl.BlockSpec((2, RB, 128), lambda r: (0, r, 0)),
        pl.BlockSpec((1, RB, 128), lambda r: (0, r, 0)),
        pl.BlockSpec((2, RB, 16), lambda r: (0, r, 0)),
        pl.BlockSpec((1, 128), lambda r: (0, 0)),
    ],
    out_specs=pl.BlockSpec((RB, 128), lambda r: (r, 0)),
    out_shape=jax.ShapeDtypeStruct((N, 128), F32),
)


def kernel(x, edge_index, W1, b1, W2, b2, W3, b3, W4, b4, W5, b5, W6, b6,
           g1, be1, g2, be2, g3, be3, g4, be4, g5, be5):
  ei = edge_index.astype(jnp.int32)
  npad = E_PAD - E_RAW
  src2 = jnp.concatenate(
      [ei[0], jnp.zeros((npad,), jnp.int32)]).reshape(EC, 1, K)
  dst2 = jnp.concatenate(
      [ei[1], jnp.full((npad,), DUMP_ROW, jnp.int32)]).reshape(EC, 1, K)
  ed3 = jnp.concatenate([src2, dst2], axis=1)
  zeros128 = jnp.zeros((RPT, 128), F32)
  deg16 = _agg_full(jnp.ones((1, N, 128), F32), ed3, zeros128)[:, :, :16]

  two = lambda v: v.reshape(2, 128)
  w_halves = lambda W: W.reshape(256, -1, 128).transpose(1, 0, 2)

  bs = [two(b1), two(b2), two(b3), two(b4), two(b5)]
  gs = [two(g1), two(g2), two(g3), two(g4), two(g5)]
  bes = [two(be1), two(be2), two(be3), two(be4), two(be5)]
  Ws = [w_halves(W2), w_halves(W3), w_halves(W4), w_halves(W5), w_halves(W6)]

  M = _m1(x, W1, deg16)
  for l in range(5):
    S = _agg_half(M, ed3, zeros128)
    st = _stats(S, M, deg16, bs[l])
    fused = _fused2 if l < 4 else _fused1
    M = fused(S, M, deg16, bs[l], st, gs[l], bes[l], Ws[l])
  S6 = _agg_full(M, ed3, zeros128)
  return _final(S6, M, deg16, b6.reshape(1, 128))


# speedup vs baseline: 12.4744x; 2.0957x over previous
import functools

import jax
import jax.numpy as jnp
from jax import lax
from jax.experimental import pallas as pl
from jax.experimental.pallas import tpu as pltpu
from jax.experimental.pallas import tpu_sc as plsc

N = 10000
F32 = jnp.float32
EPS = 1e-5
NC, NS = 2, 16
RPT = 632
NACC = NS * RPT
RPT_LAST = N - (NS - 1) * RPT
DUMP_ROW = 10008
E_RAW = 320000
K = 64
EC = 5024
EC_CAP = 5056
E_PAD = EC_CAP * K
IG = 32
NBUF = 4
RB = 2000
NRB = N // RB

_MESH = plsc.VectorSubcoreMesh(
    core_axis_name="c", subcore_axis_name="s", num_cores=NC, num_subcores=NS)


def _make_agg(split_edges):
  nch = EC // (NC * NS) if split_edges else EC // NS
  ngf, ntail = divmod(nch, IG)

  @functools.partial(
      pl.kernel,
      out_type=jax.ShapeDtypeStruct((NC, N, 128), F32),
      mesh=_MESH,
      scratch_types=[
          pltpu.VMEM((IG, 2, K), jnp.int32),
          [pltpu.VMEM((K, 128), F32) for _ in range(NBUF)],
          pltpu.VMEM_SHARED((NACC, 128), F32),
          [pltpu.SemaphoreType.DMA for _ in range(2 * NBUF)],
      ],
  )
  def agg(m, ed3, zeros, out, ed_v, sts, acc, sems):
    c = lax.axis_index("c")
    s = lax.axis_index("s")
    row0 = s * RPT
    pltpu.sync_copy(zeros, acc.at[pl.ds(row0, RPT)])
    if split_edges:
      cb = (c * NS + s) * nch
      g = c * 0
    else:
      cb = s * nch
      g = c
    plsc.subcore_barrier()

    bufs = [(sts[i], sems[2 * i], sems[2 * i + 1]) for i in range(NBUF)]

    def run_group(base, n):
      pltpu.sync_copy(ed3.at[pl.ds(base, IG)], ed_v)
      gd = [None] * n
      sd = [None] * n
      for i in range(min(NBUF, n)):
        st, gs, _ = bufs[i % NBUF]
        gd[i] = pltpu.async_copy(m.at[g].at[ed_v.at[i, 0]], st, gs)
      for i in range(n):
        st, gs, ss = bufs[i % NBUF]
        gd[i].wait()
        sd[i] = pltpu.async_copy(st, acc.at[ed_v.at[i, 1]], ss, add=True)
        if i + NBUF < n:
          sd[i].wait()
          gd[i + NBUF] = pltpu.async_copy(m.at[g].at[ed_v.at[i + NBUF, 0]],
                                          st, gs)
      for i in range(max(0, n - NBUF), n):
        sd[i].wait()

    @pl.loop(0, ngf)
    def _(jj):
      run_group(cb + jj * IG, IG)

    if ntail:
      run_group(cb + ngf * IG, ntail)

    plsc.subcore_barrier()

    @pl.when(s < NS - 1)
    def _():
      pltpu.sync_copy(acc.at[pl.ds(row0, RPT)], out.at[c, pl.ds(row0, RPT)])

    @pl.when(s == NS - 1)
    def _():
      pltpu.sync_copy(acc.at[pl.ds(row0, RPT_LAST)],
                      out.at[c, pl.ds(row0, RPT_LAST)])

  return agg


_agg_half = _make_agg(False)
_agg_full = _make_agg(True)

def _dinv_of(d_ref):
  deg = d_ref[0][:, :1] + d_ref[1][:, :1]
  return lax.rsqrt(deg + 1.0)


def _dot(a, b):
  return jnp.dot(a, b, preferred_element_type=F32,
                 precision=lax.Precision.HIGHEST)


def _m1_body(x_ref, w_ref, d_ref, o_ref):
  dinv = _dinv_of(d_ref)
  o_ref[...] = (_dot(x_ref[...], w_ref[...]) * dinv)[None]


_m1 = pl.pallas_call(
    _m1_body,
    grid=(2, NRB),
    in_specs=[
        pl.BlockSpec((RB, 128), lambda c, r: (r, 0)),
        pl.BlockSpec((128, 128), lambda c, r: (0, c)),
        pl.BlockSpec((2, RB, 16), lambda c, r: (0, r, 0)),
    ],
    out_specs=pl.BlockSpec((1, RB, 128), lambda c, r: (c, r, 0)),
    out_shape=jax.ShapeDtypeStruct((2, N, 128), F32),
)


def _stats_body(s_ref, mp_ref, d_ref, b_ref, o_ref):
  r = pl.program_id(0)
  dinv = _dinv_of(d_ref)
  sums, sqs = [], []
  for ch in range(2):
    z = dinv * (s_ref[ch] + mp_ref[ch]) + b_ref[ch]
    sums.append(jnp.sum(z, axis=0))
    sqs.append(jnp.sum(z * z, axis=0))
  sm = jnp.stack(sums)
  sq = jnp.stack(sqs)

  @pl.when(r == 0)
  def _():
    o_ref[0] = sm
    o_ref[1] = sq

  @pl.when(r > 0)
  def _():
    o_ref[0] += sm
    o_ref[1] += sq

  @pl.when(r == NRB - 1)
  def _():
    mean = o_ref[0] / N
    o_ref[0] = mean
    o_ref[1] = o_ref[1] / N - mean * mean


_stats = pl.pallas_call(
    _stats_body,
    grid=(NRB,),
    in_specs=[
        pl.BlockSpec((2, RB, 128), lambda r: (0, r, 0)),
        pl.BlockSpec((2, RB, 128), lambda r: (0, r, 0)),
        pl.BlockSpec((2, RB, 16), lambda r: (0, r, 0)),
        pl.BlockSpec((2, 128), lambda r: (0, 0)),
    ],
    out_specs=pl.BlockSpec((2, 2, 128), lambda r: (0, 0, 0)),
    out_shape=jax.ShapeDtypeStruct((2, 2, 128), F32),
)


def _fused_body(s_ref, mp_ref, d_ref, b_ref, st_ref, g_ref, be_ref, w_ref,
                o_ref):
  dinv = _dinv_of(d_ref)
  w = w_ref[0]
  acc = None
  for ch in range(2):
    z = dinv * (s_ref[ch] + mp_ref[ch]) + b_ref[ch]
    alpha = lax.rsqrt(st_ref[1, ch] + EPS) * g_ref[ch]
    h = jnp.maximum((z - st_ref[0, ch]) * alpha + be_ref[ch], 0.0) * dinv
    p = _dot(h, w[ch * 128:(ch + 1) * 128])
    acc = p if acc is None else acc + p
  o_ref[...] = acc[None]


def _make_fused(h_out):
  return pl.pallas_call(
      _fused_body,
      grid=(h_out, NRB),
      in_specs=[
          pl.BlockSpec((2, RB, 128), lambda c, r: (0, r, 0)),
          pl.BlockSpec((2, RB, 128), lambda c, r: (0, r, 0)),
          pl.BlockSpec((2, RB, 16), lambda c, r: (0, r, 0)),
          pl.BlockSpec((2, 128), lambda c, r: (0, 0)),
          pl.BlockSpec((2, 2, 128), lambda c, r: (0, 0, 0)),
          pl.BlockSpec((2, 128), lambda c, r: (0, 0)),
          pl.BlockSpec((2, 128), lambda c, r: (0, 0)),
          pl.BlockSpec((1, 256, 128), lambda c, r: (c, 0, 0)),
      ],
      out_specs=pl.BlockSpec((1, RB, 128), lambda c, r: (c, r, 0)),
      out_shape=jax.ShapeDtypeStruct((h_out, N, 128), F32),
  )


_fused2 = _make_fused(2)
_fused1 = _make_fused(1)


def _final_body(s_ref, m6_ref, d_ref, b_ref, o_ref):
  dinv = _dinv_of(d_ref)
  o_ref[...] = dinv * (s_ref[0] + s_ref[1] + m6_ref[0]) + b_ref[...]


_final = pl.pallas_call(
    _final_body,
    grid=(NRB,),
    in_specs=[
        pl.BlockSpec((2, RB, 128), lambda r: (0, r, 0)),
        pl.BlockSpec((1, RB, 128), lambda r: (0, r, 0)),
        pl.BlockSpec((2, RB, 16), lambda r: (0, r, 0)),
        pl.BlockSpec((1, 128), lambda r: (0, 0)),
    ],
    out_specs=pl.BlockSpec((RB, 128), lambda r: (r, 0)),
    out_shape=jax.ShapeDtypeStruct((N, 128), F32),
)


def kernel(x, edge_index, W1, b1, W2, b2, W3, b3, W4, b4, W5, b5, W6, b6,
           g1, be1, g2, be2, g3, be3, g4, be4, g5, be5):
  ei = edge_index.astype(jnp.int32)
  npad = E_PAD - E_RAW
  src2 = jnp.concatenate(
      [ei[0], jnp.zeros((npad,), jnp.int32)]).reshape(EC_CAP, 1, K)
  dst2 = jnp.concatenate(
      [ei[1], jnp.full((npad,), DUMP_ROW, jnp.int32)]).reshape(EC_CAP, 1, K)
  ed3 = jnp.concatenate([src2, dst2], axis=1)
  zeros128 = jnp.zeros((RPT, 128), F32)
  deg16 = _agg_full(jnp.ones((1, N, 128), F32), ed3, zeros128)[:, :, :16]

  two = lambda v: v.reshape(2, 128)
  w_halves = lambda W: W.reshape(256, -1, 128).transpose(1, 0, 2)

  bs = [two(b1), two(b2), two(b3), two(b4), two(b5)]
  gs = [two(g1), two(g2), two(g3), two(g4), two(g5)]
  bes = [two(be1), two(be2), two(be3), two(be4), two(be5)]
  Ws = [w_halves(W2), w_halves(W3), w_halves(W4), w_halves(W5), w_halves(W6)]

  M = _m1(x, W1, deg16)
  for l in range(5):
    S = _agg_half(M, ed3, zeros128)
    st = _stats(S, M, deg16, bs[l])
    fused = _fused2 if l < 4 else _fused1
    M = fused(S, M, deg16, bs[l], st, gs[l], bes[l], Ws[l])
  S6 = _agg_full(M, ed3, zeros128)
  return _final(S6, M, deg16, b6.reshape(1, 128))


# speedup vs baseline: 15.2129x; 1.2195x over previous
import functools

import jax
import jax.numpy as jnp
from jax import lax
from jax.experimental import pallas as pl
from jax.experimental.pallas import tpu as pltpu
from jax.experimental.pallas import tpu_sc as plsc

N = 10000
F32 = jnp.float32
EPS = 1e-5
NC, NS = 2, 16
RPT = 632
NACC = NS * RPT
RPT_LAST = N - (NS - 1) * RPT
DUMP_ROW = 10008
E_RAW = 320000
K = 32
EC = 10016
EC_CAP = 10080
E_PAD = EC_CAP * K
IG = 64
NBUF = 8
RB = 2000
NRB = N // RB

_MESH = plsc.VectorSubcoreMesh(
    core_axis_name="c", subcore_axis_name="s", num_cores=NC, num_subcores=NS)


def _make_agg(split_edges):
  nch = EC // (NC * NS) if split_edges else EC // NS
  ngf, ntail = divmod(nch, IG)

  @functools.partial(
      pl.kernel,
      out_type=jax.ShapeDtypeStruct((NC, N, 128), F32),
      mesh=_MESH,
      scratch_types=[
          pltpu.VMEM((IG, 2, K), jnp.int32),
          [pltpu.VMEM((K, 128), F32) for _ in range(NBUF)],
          pltpu.VMEM_SHARED((NACC, 128), F32),
          [pltpu.SemaphoreType.DMA for _ in range(2 * NBUF)],
      ],
  )
  def agg(m, ed3, zeros, out, ed_v, sts, acc, sems):
    c = lax.axis_index("c")
    s = lax.axis_index("s")
    row0 = s * RPT
    pltpu.sync_copy(zeros, acc.at[pl.ds(row0, RPT)])
    if split_edges:
      cb = (c * NS + s) * nch
      g = c * 0
    else:
      cb = s * nch
      g = c
    plsc.subcore_barrier()

    bufs = [(sts[i], sems[2 * i], sems[2 * i + 1]) for i in range(NBUF)]

    def run_group(base, n):
      pltpu.sync_copy(ed3.at[pl.ds(base, IG)], ed_v)
      gd = [None] * n
      sd = [None] * n
      for i in range(min(NBUF, n)):
        st, gs, _ = bufs[i % NBUF]
        gd[i] = pltpu.async_copy(m.at[g].at[ed_v.at[i, 0]], st, gs)
      for i in range(n):
        st, gs, ss = bufs[i % NBUF]
        gd[i].wait()
        sd[i] = pltpu.async_copy(st, acc.at[ed_v.at[i, 1]], ss, add=True)
        if i + NBUF < n:
          sd[i].wait()
          gd[i + NBUF] = pltpu.async_copy(m.at[g].at[ed_v.at[i + NBUF, 0]],
                                          st, gs)
      for i in range(max(0, n - NBUF), n):
        sd[i].wait()

    @pl.loop(0, ngf)
    def _(jj):
      run_group(cb + jj * IG, IG)

    if ntail:
      run_group(cb + ngf * IG, ntail)

    plsc.subcore_barrier()

    @pl.when(s < NS - 1)
    def _():
      pltpu.sync_copy(acc.at[pl.ds(row0, RPT)], out.at[c, pl.ds(row0, RPT)])

    @pl.when(s == NS - 1)
    def _():
      pltpu.sync_copy(acc.at[pl.ds(row0, RPT_LAST)],
                      out.at[c, pl.ds(row0, RPT_LAST)])

  return agg


_agg_half = _make_agg(False)
_agg_full = _make_agg(True)

def _dinv_of(d_ref):
  deg = d_ref[0][:, :1] + d_ref[1][:, :1]
  return lax.rsqrt(deg + 1.0)


def _dot(a, b):
  return jnp.dot(a, b, preferred_element_type=F32,
                 precision=lax.Precision.HIGHEST)


def _m1_body(x_ref, w_ref, d_ref, o_ref):
  dinv = _dinv_of(d_ref)
  o_ref[...] = (_dot(x_ref[...], w_ref[...]) * dinv)[None]


_m1 = pl.pallas_call(
    _m1_body,
    grid=(2, NRB),
    in_specs=[
        pl.BlockSpec((RB, 128), lambda c, r: (r, 0)),
        pl.BlockSpec((128, 128), lambda c, r: (0, c)),
        pl.BlockSpec((2, RB, 16), lambda c, r: (0, r, 0)),
    ],
    out_specs=pl.BlockSpec((1, RB, 128), lambda c, r: (c, r, 0)),
    out_shape=jax.ShapeDtypeStruct((2, N, 128), F32),
)


def _stats_body(s_ref, mp_ref, d_ref, b_ref, o_ref):
  r = pl.program_id(0)
  dinv = _dinv_of(d_ref)
  sums, sqs = [], []
  for ch in range(2):
    z = dinv * (s_ref[ch] + mp_ref[ch]) + b_ref[ch]
    sums.append(jnp.sum(z, axis=0))
    sqs.append(jnp.sum(z * z, axis=0))
  sm = jnp.stack(sums)
  sq = jnp.stack(sqs)

  @pl.when(r == 0)
  def _():
    o_ref[0] = sm
    o_ref[1] = sq

  @pl.when(r > 0)
  def _():
    o_ref[0] += sm
    o_ref[1] += sq

  @pl.when(r == NRB - 1)
  def _():
    mean = o_ref[0] / N
    o_ref[0] = mean
    o_ref[1] = o_ref[1] / N - mean * mean


_stats = pl.pallas_call(
    _stats_body,
    grid=(NRB,),
    in_specs=[
        pl.BlockSpec((2, RB, 128), lambda r: (0, r, 0)),
        pl.BlockSpec((2, RB, 128), lambda r: (0, r, 0)),
        pl.BlockSpec((2, RB, 16), lambda r: (0, r, 0)),
        pl.BlockSpec((2, 128), lambda r: (0, 0)),
    ],
    out_specs=pl.BlockSpec((2, 2, 128), lambda r: (0, 0, 0)),
    out_shape=jax.ShapeDtypeStruct((2, 2, 128), F32),
)


def _fused_body(s_ref, mp_ref, d_ref, b_ref, st_ref, g_ref, be_ref, w_ref,
                o_ref):
  dinv = _dinv_of(d_ref)
  w = w_ref[0]
  acc = None
  for ch in range(2):
    z = dinv * (s_ref[ch] + mp_ref[ch]) + b_ref[ch]
    alpha = lax.rsqrt(st_ref[1, ch] + EPS) * g_ref[ch]
    h = jnp.maximum((z - st_ref[0, ch]) * alpha + be_ref[ch], 0.0) * dinv
    p = _dot(h, w[ch * 128:(ch + 1) * 128])
    acc = p if acc is None else acc + p
  o_ref[...] = acc[None]


def _make_fused(h_out):
  return pl.pallas_call(
      _fused_body,
      grid=(h_out, NRB),
      in_specs=[
          pl.BlockSpec((2, RB, 128), lambda c, r: (0, r, 0)),
          pl.BlockSpec((2, RB, 128), lambda c, r: (0, r, 0)),
          pl.BlockSpec((2, RB, 16), lambda c, r: (0, r, 0)),
          pl.BlockSpec((2, 128), lambda c, r: (0, 0)),
          pl.BlockSpec((2, 2, 128), lambda c, r: (0, 0, 0)),
          pl.BlockSpec((2, 128), lambda c, r: (0, 0)),
          pl.BlockSpec((2, 128), lambda c, r: (0, 0)),
          pl.BlockSpec((1, 256, 128), lambda c, r: (c, 0, 0)),
      ],
      out_specs=pl.BlockSpec((1, RB, 128), lambda c, r: (c, r, 0)),
      out_shape=jax.ShapeDtypeStruct((h_out, N, 128), F32),
  )


_fused2 = _make_fused(2)
_fused1 = _make_fused(1)


def _final_body(s_ref, m6_ref, d_ref, b_ref, o_ref):
  dinv = _dinv_of(d_ref)
  o_ref[...] = dinv * (s_ref[0] + s_ref[1] + m6_ref[0]) + b_ref[...]


_final = pl.pallas_call(
    _final_body,
    grid=(NRB,),
    in_specs=[
        pl.BlockSpec((2, RB, 128), lambda r: (0, r, 0)),
        pl.BlockSpec((1, RB, 128), lambda r: (0, r, 0)),
        pl.BlockSpec((2, RB, 16), lambda r: (0, r, 0)),
        pl.BlockSpec((1, 128), lambda r: (0, 0)),
    ],
    out_specs=pl.BlockSpec((RB, 128), lambda r: (r, 0)),
    out_shape=jax.ShapeDtypeStruct((N, 128), F32),
)


def kernel(x, edge_index, W1, b1, W2, b2, W3, b3, W4, b4, W5, b5, W6, b6,
           g1, be1, g2, be2, g3, be3, g4, be4, g5, be5):
  ei = edge_index.astype(jnp.int32)
  npad = E_PAD - E_RAW
  src2 = jnp.concatenate(
      [ei[0], jnp.zeros((npad,), jnp.int32)]).reshape(EC_CAP, 1, K)
  dst2 = jnp.concatenate(
      [ei[1], jnp.full((npad,), DUMP_ROW, jnp.int32)]).reshape(EC_CAP, 1, K)
  ed3 = jnp.concatenate([src2, dst2], axis=1)
  zeros128 = jnp.zeros((RPT, 128), F32)
  deg16 = _agg_full(jnp.ones((1, N, 128), F32), ed3, zeros128)[:, :, :16]

  two = lambda v: v.reshape(2, 128)
  w_halves = lambda W: W.reshape(256, -1, 128).transpose(1, 0, 2)

  bs = [two(b1), two(b2), two(b3), two(b4), two(b5)]
  gs = [two(g1), two(g2), two(g3), two(g4), two(g5)]
  bes = [two(be1), two(be2), two(be3), two(be4), two(be5)]
  Ws = [w_halves(W2), w_halves(W3), w_halves(W4), w_halves(W5), w_halves(W6)]

  M = _m1(x, W1, deg16)
  for l in range(5):
    S = _agg_half(M, ed3, zeros128)
    st = _stats(S, M, deg16, bs[l])
    fused = _fused2 if l < 4 else _fused1
    M = fused(S, M, deg16, bs[l], st, gs[l], bes[l], Ws[l])
  S6 = _agg_full(M, ed3, zeros128)
  return _final(S6, M, deg16, b6.reshape(1, 128))
